# Initial kernel scaffold; baseline (speedup 1.0000x reference)
import jax
import jax.numpy as jnp
from jax.experimental import pallas as pl


def kernel(x, edge_index, edge_attr, batch, We, be, Wf0, bf0, Ws0, bs0, g0, t0, Wf1, bf1, Ws1, bs1, g1, t1, Wf2, bf2, Ws2, bs2, g2, t2, W1, b1, W2, b2):
    raise NotImplementedError("write your pallas kernel here")


# speedup vs baseline: 1.9189x; 1.9189x over previous
import functools

import jax
import jax.numpy as jnp
from jax import lax
from jax.experimental import pallas as pl
from jax.experimental.pallas import tpu as pltpu
from jax.experimental.pallas import tpu_sc as plsc

NN = 50000
EE = 800000
BB = 256
DIN = 128
DH = 64
BINS = 40
HFEA = 128

NW = 32
CH = 128
EP = 819200
NPAD = 51200
DUMMY = NN


def _softplus(x):
    return jnp.maximum(x, 0.0) + jnp.log1p(jnp.exp(-jnp.abs(x)))


def _embed_body(x_ref, we_ref, be_ref, o_ref):
    o_ref[...] = (
        jnp.dot(x_ref[...], we_ref[...], preferred_element_type=jnp.float32)
        + be_ref[...]
    )


def _embed(x, We, be2):
    blk = 2000
    grid = NN // blk
    return pl.pallas_call(
        _embed_body,
        grid=(grid,),
        in_specs=[
            pl.BlockSpec((blk, DIN), lambda i: (i, 0)),
            pl.BlockSpec((DIN, DH), lambda i: (0, 0)),
            pl.BlockSpec((1, DH), lambda i: (0, 0)),
        ],
        out_specs=pl.BlockSpec((blk, DH), lambda i: (i, 0)),
        out_shape=jax.ShapeDtypeStruct((NN, DH), jnp.float32),
    )(x, We, be2)


def _proj_body(h_ref, wfd_ref, wsd_ref, wfs_ref, wss_ref, pd_ref, ps_ref):
    h = h_ref[...]
    dot = functools.partial(jnp.dot, preferred_element_type=jnp.float32)
    pd_ref[...] = jnp.concatenate([dot(h, wfd_ref[...]), dot(h, wsd_ref[...])], axis=1)
    ps_ref[...] = jnp.concatenate([dot(h, wfs_ref[...]), dot(h, wss_ref[...])], axis=1)


def _project(h, Wf, Ws):
    blk = 2000
    grid = NN // blk
    wmat = pl.BlockSpec((DH, DH), lambda i: (0, 0))
    return pl.pallas_call(
        _proj_body,
        grid=(grid,),
        in_specs=[pl.BlockSpec((blk, DH), lambda i: (i, 0)), wmat, wmat, wmat, wmat],
        out_specs=[
            pl.BlockSpec((blk, 2 * DH), lambda i: (i, 0)),
            pl.BlockSpec((blk, 2 * DH), lambda i: (i, 0)),
        ],
        out_shape=[
            jax.ShapeDtypeStruct((NN, 2 * DH), jnp.float32),
            jax.ShapeDtypeStruct((NN, 2 * DH), jnp.float32),
        ],
    )(h, Wf[:DH], Ws[:DH], Wf[DH:2 * DH], Ws[DH:2 * DH])


def _edge_body(gd_ref, gs_ref, ea_ref, wfe_ref, bf_ref, wse_ref, bs_ref,
               m2_ref):
    ea = ea_ref[...]
    d = jnp.sqrt(jnp.sum(ea * ea, axis=1, keepdims=True))
    step = 8.0 / (BINS - 1)
    centers = lax.broadcasted_iota(jnp.int32, (1, BINS), 1).astype(jnp.float32) * step
    g = 1.0 / (step * step)
    e = jnp.exp(-g * (d - centers) ** 2)
    gd = gd_ref[...]
    gs = gs_ref[...]
    dot = functools.partial(jnp.dot, preferred_element_type=jnp.float32)
    pre_f = gd[:, :DH] + gs[:, :DH] + dot(e, wfe_ref[...]) + bf_ref[...]
    pre_s = gd[:, DH:] + gs[:, DH:] + dot(e, wse_ref[...]) + bs_ref[...]
    m = (1.0 / (1.0 + jnp.exp(-pre_f))) * _softplus(pre_s)
    blk = m.shape[0]
    m2_ref[...] = jnp.concatenate([m[:blk // 2], m[blk // 2:]], axis=1)


def _edge_gate(gd, gs, eap, Wf, bf2, Ws, bs2):
    blk = 4096
    grid = EP // blk
    emat = pl.BlockSpec((BINS, DH), lambda i: (0, 0))
    bvec = pl.BlockSpec((1, DH), lambda i: (0, 0))
    return pl.pallas_call(
        _edge_body,
        grid=(grid,),
        in_specs=[
            pl.BlockSpec((blk, 2 * DH), lambda i: (i, 0)),
            pl.BlockSpec((blk, 2 * DH), lambda i: (i, 0)),
            pl.BlockSpec((blk, 4), lambda i: (i, 0)),
            emat, bvec, emat, bvec,
        ],
        out_specs=pl.BlockSpec((blk // 2, 128), lambda i: (i, 0)),
        out_shape=jax.ShapeDtypeStruct((EP // 2, 128), jnp.float32),
    )(gd, gs, eap, Wf[2 * DH:], bf2, Ws[2 * DH:], bs2)


def _bn_stats_body(a_ref, o_ref, acc):
    i = pl.program_id(0)

    @pl.when(i == 0)
    def _():
        acc[...] = jnp.zeros_like(acc)

    a = a_ref[...]
    acc[0:1, :] += jnp.sum(a, axis=0, keepdims=True)
    acc[1:2, :] += jnp.sum(a * a, axis=0, keepdims=True)

    @pl.when(i == pl.num_programs(0) - 1)
    def _():
        o_ref[...] = jnp.zeros_like(o_ref)
        o_ref[0:2, :] = acc[...]


def _bn_apply_body(a_ref, h_ref, s_ref, g_ref, t_ref, o_ref):
    s = s_ref[...]
    mu = s[0:1, :] * (1.0 / NN)
    var = s[1:2, :] * (1.0 / NN) - mu * mu
    y = (a_ref[...] - mu) * lax.rsqrt(var + 1e-5) * g_ref[...] + t_ref[...] + h_ref[...]
    o_ref[...] = _softplus(y)


def _bn_residual(agg0, agg1, h, g2, t2):
    aggn = jnp.concatenate([agg0[:, :DH], agg1[:, :DH]], axis=0)[:NN]
    blk = 2000
    grid = NN // blk
    stats = pl.pallas_call(
        _bn_stats_body,
        grid=(grid,),
        in_specs=[pl.BlockSpec((blk, DH), lambda i: (i, 0))],
        out_specs=pl.BlockSpec((8, DH), lambda i: (0, 0)),
        out_shape=jax.ShapeDtypeStruct((8, DH), jnp.float32),
        scratch_shapes=[pltpu.VMEM((2, DH), jnp.float32)],
    )(aggn)
    return pl.pallas_call(
        _bn_apply_body,
        grid=(grid,),
        in_specs=[
            pl.BlockSpec((blk, DH), lambda i: (i, 0)),
            pl.BlockSpec((blk, DH), lambda i: (i, 0)),
            pl.BlockSpec((8, DH), lambda i: (0, 0)),
            pl.BlockSpec((1, DH), lambda i: (0, 0)),
            pl.BlockSpec((1, DH), lambda i: (0, 0)),
        ],
        out_specs=pl.BlockSpec((blk, DH), lambda i: (i, 0)),
        out_shape=jax.ShapeDtypeStruct((NN, DH), jnp.float32),
    )(aggn, h, stats, g2, t2)


def _pool_body(h_ref, b_ref, w1_ref, b1_ref, w2_ref, b2_ref, o_ref, acc):
    i = pl.program_id(0)

    @pl.when(i == 0)
    def _():
        acc[...] = jnp.zeros_like(acc)

    bvec = b_ref[0, 0, :]
    onehot = (bvec[:, None] == lax.broadcasted_iota(jnp.int32, (bvec.shape[0], BB), 1)).astype(jnp.float32)
    acc[...] += lax.dot_general(onehot, h_ref[...], (((0,), (0,)), ((), ())),
                                preferred_element_type=jnp.float32)

    @pl.when(i == pl.num_programs(0) - 1)
    def _():
        hid = _softplus(
            jnp.dot(acc[...], w1_ref[...], preferred_element_type=jnp.float32)
            + b1_ref[...])
        o_ref[...] = jnp.dot(hid, w2_ref[...], preferred_element_type=jnp.float32) + b2_ref[...]


def _pool_mlp(h, batch3, W1, b1_2, W2, b2_2):
    blk = 2000
    grid = NN // blk
    return pl.pallas_call(
        _pool_body,
        grid=(grid,),
        in_specs=[
            pl.BlockSpec((blk, DH), lambda i: (i, 0)),
            pl.BlockSpec((1, 1, blk), lambda i: (i, 0, 0)),
            pl.BlockSpec((DH, HFEA), lambda i: (0, 0)),
            pl.BlockSpec((1, HFEA), lambda i: (0, 0)),
            pl.BlockSpec((HFEA, 1), lambda i: (0, 0)),
            pl.BlockSpec((1, 1), lambda i: (0, 0)),
        ],
        out_specs=pl.BlockSpec((BB, 1), lambda i: (0, 0)),
        out_shape=jax.ShapeDtypeStruct((BB, 1), jnp.float32),
        scratch_shapes=[pltpu.VMEM((BB, DH), jnp.float32)],
    )(h, batch3, W1, b1_2, W2, b2_2)


NCHW = EP // 16 // CH


def _sc_mesh():
    return plsc.VectorSubcoreMesh(core_axis_name="c", subcore_axis_name="s",
                                  num_cores=2, num_subcores=16)


_SC_PARAMS = pltpu.CompilerParams(use_tc_tiling_on_sc=False)


GB = 2
NBATCHG = NCHW // GB


def _gather_flow(tab_hbm, idx3_hbm, out_hbm, lane, idx_v, rows, gsem, wsem):
    base = lane * (NCHW * CH)
    pltpu.sync_copy(idx3_hbm.at[lane], idx_v)

    def g_issue(cc, slot, sem):
        pltpu.async_copy(tab_hbm.at[idx_v.at[cc]], rows.at[slot], sem)

    def g_wait(slot, sem):
        pltpu.make_async_copy(tab_hbm.at[idx_v.at[0]], rows.at[slot], sem).wait()

    def w_issue(cc, slot, sem):
        pltpu.async_copy(rows.at[slot], out_hbm.at[pl.ds(base + cc * CH, CH)], sem)

    def w_wait(slot, sem):
        pltpu.make_async_copy(rows.at[slot], out_hbm.at[pl.ds(base, CH)], sem).wait()

    for k in range(GB):
        g_issue(k, k, gsem.at[0])

    def body(b0, _):
        for hb in range(2):
            bb = 2 * b0 + hb
            nb = bb + 1

            @pl.when(bb >= 1)
            def _():
                for k in range(GB):
                    w_wait((1 - hb) * GB + k, wsem.at[1 - hb])

            @pl.when(nb < NBATCHG)
            def _():
                for k in range(GB):
                    g_issue(nb * GB + k, (1 - hb) * GB + k, gsem.at[1 - hb])

            for k in range(GB):
                g_wait(hb * GB + k, gsem.at[hb])
            for k in range(GB):
                w_issue(bb * GB + k, hb * GB + k, wsem.at[hb])
        return _

    lax.fori_loop(0, NBATCHG // 2, body, None)
    for k in range(GB):
        w_wait(GB + k, wsem.at[1])


def _sc_gather(pd, ps, dst3, src3):
    mesh = _sc_mesh()

    @functools.partial(
        pl.kernel,
        out_type=(jax.ShapeDtypeStruct((EP, 2 * DH), jnp.float32),
                  jax.ShapeDtypeStruct((EP, 2 * DH), jnp.float32)),
        mesh=mesh,
        scratch_types=[
            pltpu.VMEM((NCHW, CH), jnp.int32),
            pltpu.VMEM((2 * GB, CH, 2 * DH), jnp.float32),
            pltpu.SemaphoreType.DMA((2,)),
            pltpu.SemaphoreType.DMA((2,)),
        ],
        compiler_params=_SC_PARAMS,
    )
    def k(pd_hbm, ps_hbm, dst3_hbm, src3_hbm, gd_hbm, gs_hbm, idx_v, rows, gsem, wsem):
        wid = lax.axis_index("s") * 2 + lax.axis_index("c")
        lane = wid % 16

        @pl.when(wid < 16)
        def _():
            _gather_flow(pd_hbm, dst3_hbm, gd_hbm, lane, idx_v, rows, gsem, wsem)

        @pl.when(wid >= 16)
        def _():
            _gather_flow(ps_hbm, src3_hbm, gs_hbm, lane, idx_v, rows, gsem, wsem)

    return k(pd, ps, dst3, src3)


NHALF = NPAD // 2
ROWS_PT = NHALF // 16
ZCH = 64
SENT = -1


def _scatter_flow(m_hbm, idx3_hbm, out_hbm, tile, idxr, rows, zbuf, acc,
                  lsem, asem, zsem):
    def zrow(r, _):
        for j in range(4):
            zbuf[r, pl.ds(16 * j, 16)] = jnp.zeros((16,), jnp.float32)
        return _
    lax.fori_loop(0, ZCH, zrow, None)
    for q in range(ROWS_PT // ZCH):
        pltpu.async_copy(zbuf, acc.at[pl.ds(tile * ROWS_PT + q * ZCH, ZCH)], zsem)
    for q in range(ROWS_PT // ZCH):
        pltpu.make_async_copy(zbuf, acc.at[pl.ds(0, ZCH)], zsem).wait()
    plsc.subcore_barrier()

    mbase = tile * (NCHW * CH // 2)

    def l_issue(cc, slot, sem):
        r0 = mbase + cc * (CH // 2)
        pltpu.async_copy(m_hbm.at[pl.ds(r0, CH // 2), pl.ds(0, DH)],
                         rows.at[slot].at[pl.ds(0, CH // 2)], sem)
        pltpu.async_copy(m_hbm.at[pl.ds(r0, CH // 2), pl.ds(DH, DH)],
                         rows.at[slot].at[pl.ds(CH // 2, CH // 2)], sem)
        pltpu.async_copy(idx3_hbm.at[tile].at[cc], idxr.at[slot], sem)

    def l_wait(slot, sem):
        pltpu.make_async_copy(m_hbm.at[pl.ds(mbase, CH // 2), pl.ds(0, DH)],
                              rows.at[slot].at[pl.ds(0, CH // 2)], sem).wait()
        pltpu.make_async_copy(m_hbm.at[pl.ds(mbase, CH // 2), pl.ds(DH, DH)],
                              rows.at[slot].at[pl.ds(CH // 2, CH // 2)], sem).wait()
        pltpu.make_async_copy(idx3_hbm.at[tile].at[0], idxr.at[slot], sem).wait()

    def a_issue(slot, sem):
        pltpu.async_copy(rows.at[slot],
                         acc.at[plsc.Indices(idxr.at[slot], ignored_value=SENT)],
                         sem, add=True)

    def a_wait(slot, sem):
        pltpu.make_async_copy(rows.at[slot],
                              acc.at[plsc.Indices(idxr.at[slot], ignored_value=SENT)],
                              sem).wait()

    l_issue(0, 0, lsem.at[0])

    def body(b0, _):
        for hb in range(2):
            bb = 2 * b0 + hb
            nb = bb + 1

            @pl.when(bb >= 1)
            def _():
                a_wait(1 - hb, asem.at[1 - hb])

            @pl.when(nb < NCHW)
            def _():
                l_issue(nb, 1 - hb, lsem.at[1 - hb])

            l_wait(hb, lsem.at[hb])
            a_issue(hb, asem.at[hb])
        return _

    lax.fori_loop(0, NCHW // 2, body, None)
    a_wait(1, asem.at[1])
    plsc.subcore_barrier()
    pltpu.sync_copy(acc.at[pl.ds(tile * ROWS_PT, ROWS_PT)],
                    out_hbm.at[pl.ds(tile * ROWS_PT, ROWS_PT), pl.ds(0, DH)])


def _sc_scatter(m2, idxc0, idxc1):
    mesh = _sc_mesh()

    @functools.partial(
        pl.kernel,
        out_type=(jax.ShapeDtypeStruct((NHALF, 128), jnp.float32),
                  jax.ShapeDtypeStruct((NHALF, 128), jnp.float32)),
        mesh=mesh,
        scratch_types=[
            pltpu.VMEM((2, CH), jnp.int32),
            pltpu.VMEM((2, CH, DH), jnp.float32),
            pltpu.VMEM((ZCH, DH), jnp.float32),
            pltpu.VMEM_SHARED((NHALF, DH), jnp.float32),
            pltpu.SemaphoreType.DMA((2,)),
            pltpu.SemaphoreType.DMA((2,)),
            pltpu.SemaphoreType.DMA,
        ],
        compiler_params=_SC_PARAMS,
    )
    def k(m2_hbm, idxc0_hbm, idxc1_hbm, agg0_hbm, agg1_hbm,
          idxr, rows, zbuf, acc, lsem, asem, zsem):
        c = lax.axis_index("c")
        tile = lax.axis_index("s")

        @pl.when(c == 0)
        def _():
            _scatter_flow(m2_hbm, idxc0_hbm, agg0_hbm, tile, idxr, rows, zbuf,
                          acc, lsem, asem, zsem)

        @pl.when(c == 1)
        def _():
            _scatter_flow(m2_hbm, idxc1_hbm, agg1_hbm, tile, idxr, rows, zbuf,
                          acc, lsem, asem, zsem)

    return k(m2, idxc0, idxc1)


def _gather_edges(pd, ps, dst3, src3):
    return _sc_gather(pd, ps, dst3, src3)


def _scatter_edges(m2, idxc0, idxc1):
    return _sc_scatter(m2, idxc0, idxc1)


def kernel(x, edge_index, edge_attr, batch, We, be, Wf0, bf0, Ws0, bs0, g0, t0,
           Wf1, bf1, Ws1, bs1, g1, t1, Wf2, bf2, Ws2, bs2, g2, t2, W1, b1, W2, b2):
    src = edge_index[0]
    dst = edge_index[1]
    pad = EP - EE
    src3 = jnp.pad(src, (0, pad)).reshape(16, NCHW, CH)
    dst3 = jnp.pad(dst, (0, pad)).reshape(16, NCHW, CH)
    dstp = jnp.pad(dst, (0, pad), constant_values=DUMMY)
    dperm = dstp.reshape(EP // 4096, 2, 32, 64).transpose(0, 2, 1, 3).reshape(EP)
    idxc0 = jnp.where(dperm < NHALF, dperm, SENT).astype(jnp.int32).reshape(16, NCHW, CH)
    idxc1 = jnp.where(dperm >= NHALF, dperm - NHALF, SENT).astype(jnp.int32).reshape(16, NCHW, CH)
    eap = jnp.pad(edge_attr, ((0, pad), (0, 0)))
    batch3 = batch.reshape(25, 1, 2000)

    h = _embed(x, We, be.reshape(1, DH))
    for (Wf, bf, Ws, bs, g, t) in ((Wf0, bf0, Ws0, bs0, g0, t0),
                                   (Wf1, bf1, Ws1, bs1, g1, t1),
                                   (Wf2, bf2, Ws2, bs2, g2, t2)):
        pd, ps = _project(h, Wf, Ws)
        gd, gs = _gather_edges(pd, ps, dst3, src3)
        m2 = _edge_gate(gd, gs, eap, Wf, bf.reshape(1, DH), Ws, bs.reshape(1, DH))
        agg0, agg1 = _scatter_edges(m2, idxc0, idxc1)
        h = _bn_residual(agg0, agg1, h, g.reshape(1, DH), t.reshape(1, DH))

    return _pool_mlp(h, batch3, W1, b1.reshape(1, HFEA), W2, b2.reshape(1, 1))


# speedup vs baseline: 2.1937x; 1.1432x over previous
import functools

import jax
import jax.numpy as jnp
from jax import lax
from jax.experimental import pallas as pl
from jax.experimental.pallas import tpu as pltpu
from jax.experimental.pallas import tpu_sc as plsc

NN = 50000
EE = 800000
BB = 256
DIN = 128
DH = 64
BINS = 40
HFEA = 128

NW = 32
CH = 128
EP = 819200
NPAD = 51200
DUMMY = NN


def _softplus(x):
    return jnp.maximum(x, 0.0) + jnp.log1p(jnp.exp(-jnp.abs(x)))


def _embed_body(x_ref, we_ref, be_ref, o_ref):
    o_ref[...] = (
        jnp.dot(x_ref[...], we_ref[...], preferred_element_type=jnp.float32)
        + be_ref[...]
    )


def _embed(x, We, be2):
    blk = 2000
    grid = NN // blk
    return pl.pallas_call(
        _embed_body,
        grid=(grid,),
        in_specs=[
            pl.BlockSpec((blk, DIN), lambda i: (i, 0)),
            pl.BlockSpec((DIN, DH), lambda i: (0, 0)),
            pl.BlockSpec((1, DH), lambda i: (0, 0)),
        ],
        out_specs=pl.BlockSpec((blk, DH), lambda i: (i, 0)),
        out_shape=jax.ShapeDtypeStruct((NN, DH), jnp.float32),
    )(x, We, be2)


def _proj_body(h_ref, wfd_ref, wsd_ref, wfs_ref, wss_ref, pd_ref, ps_ref):
    h = h_ref[...]
    dot = functools.partial(jnp.dot, preferred_element_type=jnp.float32)
    pd_ref[...] = jnp.concatenate([dot(h, wfd_ref[...]), dot(h, wsd_ref[...])], axis=1)
    ps_ref[...] = jnp.concatenate([dot(h, wfs_ref[...]), dot(h, wss_ref[...])], axis=1)


def _project(h, Wf, Ws):
    blk = 2000
    grid = NN // blk
    wmat = pl.BlockSpec((DH, DH), lambda i: (0, 0))
    return pl.pallas_call(
        _proj_body,
        grid=(grid,),
        in_specs=[pl.BlockSpec((blk, DH), lambda i: (i, 0)), wmat, wmat, wmat, wmat],
        out_specs=[
            pl.BlockSpec((blk, 2 * DH), lambda i: (i, 0)),
            pl.BlockSpec((blk, 2 * DH), lambda i: (i, 0)),
        ],
        out_shape=[
            jax.ShapeDtypeStruct((NN, 2 * DH), jnp.float32),
            jax.ShapeDtypeStruct((NN, 2 * DH), jnp.float32),
        ],
    )(h, Wf[:DH], Ws[:DH], Wf[DH:2 * DH], Ws[DH:2 * DH])


def _edge_body(gd_ref, gs_ref, dd_ref, wfe_ref, bf_ref, wse_ref, bs_ref,
               m2_ref):
    d = jnp.sqrt(dd_ref[...])
    step = 8.0 / (BINS - 1)
    centers = lax.broadcasted_iota(jnp.int32, (1, BINS), 1).astype(jnp.float32) * step
    g = 1.0 / (step * step)
    e = jnp.exp(-g * (d - centers) ** 2)
    gd = gd_ref[...]
    gs = gs_ref[...]
    dot = functools.partial(jnp.dot, preferred_element_type=jnp.float32)
    pre_f = gd[:, :DH] + gs[:, :DH] + dot(e, wfe_ref[...]) + bf_ref[...]
    pre_s = gd[:, DH:] + gs[:, DH:] + dot(e, wse_ref[...]) + bs_ref[...]
    m = (1.0 / (1.0 + jnp.exp(-pre_f))) * _softplus(pre_s)
    blk = m.shape[0]
    m2_ref[...] = jnp.concatenate([m[:blk // 2], m[blk // 2:]], axis=1)


def _edge_gate(gd, gs, ddp, Wf, bf2, Ws, bs2):
    blk = 4096
    grid = EP // blk
    emat = pl.BlockSpec((BINS, DH), lambda i: (0, 0))
    bvec = pl.BlockSpec((1, DH), lambda i: (0, 0))
    return pl.pallas_call(
        _edge_body,
        grid=(grid,),
        in_specs=[
            pl.BlockSpec((blk, 2 * DH), lambda i: (i, 0)),
            pl.BlockSpec((blk, 2 * DH), lambda i: (i, 0)),
            pl.BlockSpec((blk, 1), lambda i: (i, 0)),
            emat, bvec, emat, bvec,
        ],
        out_specs=pl.BlockSpec((blk // 2, 128), lambda i: (i, 0)),
        out_shape=jax.ShapeDtypeStruct((EP // 2, 128), jnp.float32),
    )(gd, gs, ddp, Wf[2 * DH:], bf2, Ws[2 * DH:], bs2)


def _bn_stats_body(a_ref, o_ref, acc):
    i = pl.program_id(0)

    @pl.when(i == 0)
    def _():
        acc[...] = jnp.zeros_like(acc)

    a = a_ref[...]
    acc[0:1, :] += jnp.sum(a, axis=0, keepdims=True)
    acc[1:2, :] += jnp.sum(a * a, axis=0, keepdims=True)

    @pl.when(i == pl.num_programs(0) - 1)
    def _():
        o_ref[...] = jnp.zeros_like(o_ref)
        o_ref[0:2, :] = acc[...]


def _bn_apply_body(a_ref, h_ref, s_ref, g_ref, t_ref, o_ref):
    s = s_ref[...]
    mu = s[0:1, :] * (1.0 / NN)
    var = s[1:2, :] * (1.0 / NN) - mu * mu
    y = (a_ref[...] - mu) * lax.rsqrt(var + 1e-5) * g_ref[...] + t_ref[...] + h_ref[...]
    o_ref[...] = _softplus(y)


def _bn_residual(agg0, agg1, h, g2, t2):
    aggn = jnp.concatenate([agg0[:, :DH], agg1[:, :DH]], axis=0)[:NN]
    blk = 2000
    grid = NN // blk
    stats = pl.pallas_call(
        _bn_stats_body,
        grid=(grid,),
        in_specs=[pl.BlockSpec((blk, DH), lambda i: (i, 0))],
        out_specs=pl.BlockSpec((8, DH), lambda i: (0, 0)),
        out_shape=jax.ShapeDtypeStruct((8, DH), jnp.float32),
        scratch_shapes=[pltpu.VMEM((2, DH), jnp.float32)],
    )(aggn)
    return pl.pallas_call(
        _bn_apply_body,
        grid=(grid,),
        in_specs=[
            pl.BlockSpec((blk, DH), lambda i: (i, 0)),
            pl.BlockSpec((blk, DH), lambda i: (i, 0)),
            pl.BlockSpec((8, DH), lambda i: (0, 0)),
            pl.BlockSpec((1, DH), lambda i: (0, 0)),
            pl.BlockSpec((1, DH), lambda i: (0, 0)),
        ],
        out_specs=pl.BlockSpec((blk, DH), lambda i: (i, 0)),
        out_shape=jax.ShapeDtypeStruct((NN, DH), jnp.float32),
    )(aggn, h, stats, g2, t2)


def _pool_body(h_ref, b_ref, w1_ref, b1_ref, w2_ref, b2_ref, o_ref, acc):
    i = pl.program_id(0)

    @pl.when(i == 0)
    def _():
        acc[...] = jnp.zeros_like(acc)

    bvec = b_ref[0, 0, :]
    onehot = (bvec[:, None] == lax.broadcasted_iota(jnp.int32, (bvec.shape[0], BB), 1)).astype(jnp.float32)
    acc[...] += lax.dot_general(onehot, h_ref[...], (((0,), (0,)), ((), ())),
                                preferred_element_type=jnp.float32)

    @pl.when(i == pl.num_programs(0) - 1)
    def _():
        hid = _softplus(
            jnp.dot(acc[...], w1_ref[...], preferred_element_type=jnp.float32)
            + b1_ref[...])
        o_ref[...] = jnp.dot(hid, w2_ref[...], preferred_element_type=jnp.float32) + b2_ref[...]


def _pool_mlp(h, batch3, W1, b1_2, W2, b2_2):
    blk = 2000
    grid = NN // blk
    return pl.pallas_call(
        _pool_body,
        grid=(grid,),
        in_specs=[
            pl.BlockSpec((blk, DH), lambda i: (i, 0)),
            pl.BlockSpec((1, 1, blk), lambda i: (i, 0, 0)),
            pl.BlockSpec((DH, HFEA), lambda i: (0, 0)),
            pl.BlockSpec((1, HFEA), lambda i: (0, 0)),
            pl.BlockSpec((HFEA, 1), lambda i: (0, 0)),
            pl.BlockSpec((1, 1), lambda i: (0, 0)),
        ],
        out_specs=pl.BlockSpec((BB, 1), lambda i: (0, 0)),
        out_shape=jax.ShapeDtypeStruct((BB, 1), jnp.float32),
        scratch_shapes=[pltpu.VMEM((BB, DH), jnp.float32)],
    )(h, batch3, W1, b1_2, W2, b2_2)


NCHW = EP // 16 // CH


def _sc_mesh():
    return plsc.VectorSubcoreMesh(core_axis_name="c", subcore_axis_name="s",
                                  num_cores=2, num_subcores=16)


_SC_PARAMS = pltpu.CompilerParams(use_tc_tiling_on_sc=False)


GB = 2
NBATCHG = NCHW // GB


def _gather_flow(tab_hbm, idx3_hbm, out_hbm, lane, idx_v, rows, gsem, wsem):
    base = lane * (NCHW * CH)
    pltpu.sync_copy(idx3_hbm.at[lane], idx_v)

    def g_issue(cc, slot, sem):
        pltpu.async_copy(tab_hbm.at[idx_v.at[cc]], rows.at[slot], sem)

    def g_wait(slot, sem):
        pltpu.make_async_copy(tab_hbm.at[idx_v.at[0]], rows.at[slot], sem).wait()

    def w_issue(cc, slot, sem):
        pltpu.async_copy(rows.at[slot], out_hbm.at[pl.ds(base + cc * CH, CH)], sem)

    def w_wait(slot, sem):
        pltpu.make_async_copy(rows.at[slot], out_hbm.at[pl.ds(base, CH)], sem).wait()

    for k in range(GB):
        g_issue(k, k, gsem.at[0])

    def body(b0, _):
        for hb in range(2):
            bb = 2 * b0 + hb
            nb = bb + 1

            @pl.when(bb >= 1)
            def _():
                for k in range(GB):
                    w_wait((1 - hb) * GB + k, wsem.at[1 - hb])

            @pl.when(nb < NBATCHG)
            def _():
                for k in range(GB):
                    g_issue(nb * GB + k, (1 - hb) * GB + k, gsem.at[1 - hb])

            for k in range(GB):
                g_wait(hb * GB + k, gsem.at[hb])
            for k in range(GB):
                w_issue(bb * GB + k, hb * GB + k, wsem.at[hb])
        return _

    lax.fori_loop(0, NBATCHG // 2, body, None)
    for k in range(GB):
        w_wait(GB + k, wsem.at[1])


def _sc_gather(pd, ps, dst3, src3):
    mesh = _sc_mesh()

    @functools.partial(
        pl.kernel,
        out_type=(jax.ShapeDtypeStruct((EP, 2 * DH), jnp.float32),
                  jax.ShapeDtypeStruct((EP, 2 * DH), jnp.float32)),
        mesh=mesh,
        scratch_types=[
            pltpu.VMEM((NCHW, CH), jnp.int32),
            pltpu.VMEM((2 * GB, CH, 2 * DH), jnp.float32),
            pltpu.SemaphoreType.DMA((2,)),
            pltpu.SemaphoreType.DMA((2,)),
        ],
        compiler_params=_SC_PARAMS,
    )
    def k(pd_hbm, ps_hbm, dst3_hbm, src3_hbm, gd_hbm, gs_hbm, idx_v, rows, gsem, wsem):
        wid = lax.axis_index("s") * 2 + lax.axis_index("c")
        lane = wid % 16

        @pl.when(wid < 16)
        def _():
            _gather_flow(pd_hbm, dst3_hbm, gd_hbm, lane, idx_v, rows, gsem, wsem)

        @pl.when(wid >= 16)
        def _():
            _gather_flow(ps_hbm, src3_hbm, gs_hbm, lane, idx_v, rows, gsem, wsem)

    return k(pd, ps, dst3, src3)


NHALF = NPAD // 2
ROWS_PT = NHALF // 16
ZCH = 64
SENT = -1


def _scatter_flow(m_hbm, idx3_hbm, out_hbm, tile, idxr, rows, zbuf, acc,
                  lsem, asem, zsem):
    def zrow(r, _):
        for j in range(4):
            zbuf[r, pl.ds(16 * j, 16)] = jnp.zeros((16,), jnp.float32)
        return _
    lax.fori_loop(0, ZCH, zrow, None)
    for q in range(ROWS_PT // ZCH):
        pltpu.async_copy(zbuf, acc.at[pl.ds(tile * ROWS_PT + q * ZCH, ZCH)], zsem)
    for q in range(ROWS_PT // ZCH):
        pltpu.make_async_copy(zbuf, acc.at[pl.ds(0, ZCH)], zsem).wait()
    plsc.subcore_barrier()

    mbase = tile * (NCHW * CH // 2)

    def l_issue(cc, slot, sem):
        r0 = mbase + cc * (CH // 2)
        pltpu.async_copy(m_hbm.at[pl.ds(r0, CH // 2), pl.ds(0, DH)],
                         rows.at[slot].at[pl.ds(0, CH // 2)], sem)
        pltpu.async_copy(m_hbm.at[pl.ds(r0, CH // 2), pl.ds(DH, DH)],
                         rows.at[slot].at[pl.ds(CH // 2, CH // 2)], sem)
        pltpu.async_copy(idx3_hbm.at[tile].at[cc], idxr.at[slot], sem)

    def l_wait(slot, sem):
        pltpu.make_async_copy(m_hbm.at[pl.ds(mbase, CH // 2), pl.ds(0, DH)],
                              rows.at[slot].at[pl.ds(0, CH // 2)], sem).wait()
        pltpu.make_async_copy(m_hbm.at[pl.ds(mbase, CH // 2), pl.ds(DH, DH)],
                              rows.at[slot].at[pl.ds(CH // 2, CH // 2)], sem).wait()
        pltpu.make_async_copy(idx3_hbm.at[tile].at[0], idxr.at[slot], sem).wait()

    def a_issue(slot, sem):
        pltpu.async_copy(rows.at[slot],
                         acc.at[plsc.Indices(idxr.at[slot], ignored_value=SENT)],
                         sem, add=True)

    def a_wait(slot, sem):
        pltpu.make_async_copy(rows.at[slot],
                              acc.at[plsc.Indices(idxr.at[slot], ignored_value=SENT)],
                              sem).wait()

    l_issue(0, 0, lsem.at[0])

    def body(b0, _):
        for hb in range(2):
            bb = 2 * b0 + hb
            nb = bb + 1

            @pl.when(bb >= 1)
            def _():
                a_wait(1 - hb, asem.at[1 - hb])

            @pl.when(nb < NCHW)
            def _():
                l_issue(nb, 1 - hb, lsem.at[1 - hb])

            l_wait(hb, lsem.at[hb])
            a_issue(hb, asem.at[hb])
        return _

    lax.fori_loop(0, NCHW // 2, body, None)
    a_wait(1, asem.at[1])
    plsc.subcore_barrier()
    pltpu.sync_copy(acc.at[pl.ds(tile * ROWS_PT, ROWS_PT)],
                    out_hbm.at[pl.ds(tile * ROWS_PT, ROWS_PT), pl.ds(0, DH)])


def _sc_scatter(m2, idxc0, idxc1):
    mesh = _sc_mesh()

    @functools.partial(
        pl.kernel,
        out_type=(jax.ShapeDtypeStruct((NHALF, 128), jnp.float32),
                  jax.ShapeDtypeStruct((NHALF, 128), jnp.float32)),
        mesh=mesh,
        scratch_types=[
            pltpu.VMEM((2, CH), jnp.int32),
            pltpu.VMEM((2, CH, DH), jnp.float32),
            pltpu.VMEM((ZCH, DH), jnp.float32),
            pltpu.VMEM_SHARED((NHALF, DH), jnp.float32),
            pltpu.SemaphoreType.DMA((2,)),
            pltpu.SemaphoreType.DMA((2,)),
            pltpu.SemaphoreType.DMA,
        ],
        compiler_params=_SC_PARAMS,
    )
    def k(m2_hbm, idxc0_hbm, idxc1_hbm, agg0_hbm, agg1_hbm,
          idxr, rows, zbuf, acc, lsem, asem, zsem):
        c = lax.axis_index("c")
        tile = lax.axis_index("s")

        @pl.when(c == 0)
        def _():
            _scatter_flow(m2_hbm, idxc0_hbm, agg0_hbm, tile, idxr, rows, zbuf,
                          acc, lsem, asem, zsem)

        @pl.when(c == 1)
        def _():
            _scatter_flow(m2_hbm, idxc1_hbm, agg1_hbm, tile, idxr, rows, zbuf,
                          acc, lsem, asem, zsem)

    return k(m2, idxc0, idxc1)


def _gather_edges(pd, ps, dst3, src3):
    return _sc_gather(pd, ps, dst3, src3)


def _scatter_edges(m2, idxc0, idxc1):
    return _sc_scatter(m2, idxc0, idxc1)


def kernel(x, edge_index, edge_attr, batch, We, be, Wf0, bf0, Ws0, bs0, g0, t0,
           Wf1, bf1, Ws1, bs1, g1, t1, Wf2, bf2, Ws2, bs2, g2, t2, W1, b1, W2, b2):
    src = edge_index[0]
    dst = edge_index[1]
    pad = EP - EE
    src3 = jnp.pad(src, (0, pad)).reshape(16, NCHW, CH)
    dst3 = jnp.pad(dst, (0, pad)).reshape(16, NCHW, CH)
    dstp = jnp.pad(dst, (0, pad), constant_values=DUMMY)
    dperm = dstp.reshape(EP // 4096, 2, 32, 64).transpose(0, 2, 1, 3).reshape(EP)
    idxc0 = jnp.where(dperm < NHALF, dperm, SENT).astype(jnp.int32).reshape(16, NCHW, CH)
    idxc1 = jnp.where(dperm >= NHALF, dperm - NHALF, SENT).astype(jnp.int32).reshape(16, NCHW, CH)
    ddp = jnp.pad(jnp.sum(edge_attr * edge_attr, axis=1), (0, pad)).reshape(EP, 1)
    batch3 = batch.reshape(25, 1, 2000)

    h = _embed(x, We, be.reshape(1, DH))
    for (Wf, bf, Ws, bs, g, t) in ((Wf0, bf0, Ws0, bs0, g0, t0),
                                   (Wf1, bf1, Ws1, bs1, g1, t1),
                                   (Wf2, bf2, Ws2, bs2, g2, t2)):
        pd, ps = _project(h, Wf, Ws)
        gd, gs = _gather_edges(pd, ps, dst3, src3)
        m2 = _edge_gate(gd, gs, ddp, Wf, bf.reshape(1, DH), Ws, bs.reshape(1, DH))
        agg0, agg1 = _scatter_edges(m2, idxc0, idxc1)
        h = _bn_residual(agg0, agg1, h, g.reshape(1, DH), t.reshape(1, DH))

    return _pool_mlp(h, batch3, W1, b1.reshape(1, HFEA), W2, b2.reshape(1, 1))


# speedup vs baseline: 2.4104x; 1.0988x over previous
import functools

import jax
import jax.numpy as jnp
from jax import lax
from jax.experimental import pallas as pl
from jax.experimental.pallas import tpu as pltpu
from jax.experimental.pallas import tpu_sc as plsc

NN = 50000
EE = 800000
BB = 256
DIN = 128
DH = 64
BINS = 40
HFEA = 128

NW = 32
CH = 128
EP = 819200
NPAD = 51200
DUMMY = NN


def _softplus(x):
    return jnp.maximum(x, 0.0) + jnp.log1p(jnp.exp(-jnp.abs(x)))


def _embed_body(x_ref, we_ref, be_ref, o_ref):
    o_ref[...] = (
        jnp.dot(x_ref[...], we_ref[...], preferred_element_type=jnp.float32)
        + be_ref[...]
    )


def _embed(x, We, be2):
    blk = 2000
    grid = NN // blk
    return pl.pallas_call(
        _embed_body,
        grid=(grid,),
        in_specs=[
            pl.BlockSpec((blk, DIN), lambda i: (i, 0)),
            pl.BlockSpec((DIN, DH), lambda i: (0, 0)),
            pl.BlockSpec((1, DH), lambda i: (0, 0)),
        ],
        out_specs=pl.BlockSpec((blk, DH), lambda i: (i, 0)),
        out_shape=jax.ShapeDtypeStruct((NN, DH), jnp.float32),
    )(x, We, be2)


def _proj_body(h_ref, wfd_ref, wsd_ref, wfs_ref, wss_ref, pd_ref, ps_ref):
    h = h_ref[...]
    dot = functools.partial(jnp.dot, preferred_element_type=jnp.float32)
    pd_ref[...] = jnp.concatenate([dot(h, wfd_ref[...]), dot(h, wsd_ref[...])], axis=1)
    ps_ref[...] = jnp.concatenate([dot(h, wfs_ref[...]), dot(h, wss_ref[...])], axis=1)


def _project(h, Wf, Ws):
    blk = 2000
    grid = NN // blk
    wmat = pl.BlockSpec((DH, DH), lambda i: (0, 0))
    return pl.pallas_call(
        _proj_body,
        grid=(grid,),
        in_specs=[pl.BlockSpec((blk, DH), lambda i: (i, 0)), wmat, wmat, wmat, wmat],
        out_specs=[
            pl.BlockSpec((blk, 2 * DH), lambda i: (i, 0)),
            pl.BlockSpec((blk, 2 * DH), lambda i: (i, 0)),
        ],
        out_shape=[
            jax.ShapeDtypeStruct((NN, 2 * DH), jnp.float32),
            jax.ShapeDtypeStruct((NN, 2 * DH), jnp.float32),
        ],
    )(h, Wf[:DH], Ws[:DH], Wf[DH:2 * DH], Ws[DH:2 * DH])


def _edge_body(gd_ref, gs_ref, dd_ref, wfe_ref, bf_ref, wse_ref, bs_ref,
               m2_ref):
    d = jnp.sqrt(dd_ref[...])
    step = 8.0 / (BINS - 1)
    centers = lax.broadcasted_iota(jnp.int32, (1, BINS), 1).astype(jnp.float32) * step
    g = 1.0 / (step * step)
    e = jnp.exp(-g * (d - centers) ** 2)
    gd = gd_ref[...]
    gs = gs_ref[...]
    dot = functools.partial(jnp.dot, preferred_element_type=jnp.float32)
    pre_f = gd[:, :DH] + gs[:, :DH] + dot(e, wfe_ref[...]) + bf_ref[...]
    pre_s = gd[:, DH:] + gs[:, DH:] + dot(e, wse_ref[...]) + bs_ref[...]
    m = (1.0 / (1.0 + jnp.exp(-pre_f))) * _softplus(pre_s)
    blk = m.shape[0]
    m2_ref[...] = jnp.concatenate([m[:blk // 2], m[blk // 2:]], axis=1)


def _edge_gate(gd, gs, ddp, Wf, bf2, Ws, bs2):
    blk = 4096
    eps = gd.shape[0]
    grid = eps // blk
    emat = pl.BlockSpec((BINS, DH), lambda i: (0, 0))
    bvec = pl.BlockSpec((1, DH), lambda i: (0, 0))
    return pl.pallas_call(
        _edge_body,
        grid=(grid,),
        in_specs=[
            pl.BlockSpec((blk, 2 * DH), lambda i: (i, 0)),
            pl.BlockSpec((blk, 2 * DH), lambda i: (i, 0)),
            pl.BlockSpec((blk, 1), lambda i: (i, 0)),
            emat, bvec, emat, bvec,
        ],
        out_specs=pl.BlockSpec((blk // 2, 128), lambda i: (i, 0)),
        out_shape=jax.ShapeDtypeStruct((eps // 2, 128), jnp.float32),
    )(gd, gs, ddp, Wf[2 * DH:], bf2, Ws[2 * DH:], bs2)


def _bn_stats_body(a_ref, o_ref, acc):
    i = pl.program_id(0)

    @pl.when(i == 0)
    def _():
        acc[...] = jnp.zeros_like(acc)

    a = a_ref[...]
    acc[0:1, :] += jnp.sum(a, axis=0, keepdims=True)
    acc[1:2, :] += jnp.sum(a * a, axis=0, keepdims=True)

    @pl.when(i == pl.num_programs(0) - 1)
    def _():
        o_ref[...] = jnp.zeros_like(o_ref)
        o_ref[0:2, :] = acc[...]


def _bn_apply_body(a_ref, h_ref, s_ref, g_ref, t_ref, o_ref):
    s = s_ref[...]
    mu = s[0:1, :] * (1.0 / NN)
    var = s[1:2, :] * (1.0 / NN) - mu * mu
    y = (a_ref[...] - mu) * lax.rsqrt(var + 1e-5) * g_ref[...] + t_ref[...] + h_ref[...]
    o_ref[...] = _softplus(y)


def _bn_residual(aggs, h, g2, t2):
    a0 = sum(a[:, :DH] for a, _ in aggs)
    a1 = sum(a[:, :DH] for _, a in aggs)
    aggn = jnp.concatenate([a0, a1], axis=0)[:NN]
    blk = 2000
    grid = NN // blk
    stats = pl.pallas_call(
        _bn_stats_body,
        grid=(grid,),
        in_specs=[pl.BlockSpec((blk, DH), lambda i: (i, 0))],
        out_specs=pl.BlockSpec((8, DH), lambda i: (0, 0)),
        out_shape=jax.ShapeDtypeStruct((8, DH), jnp.float32),
        scratch_shapes=[pltpu.VMEM((2, DH), jnp.float32)],
    )(aggn)
    return pl.pallas_call(
        _bn_apply_body,
        grid=(grid,),
        in_specs=[
            pl.BlockSpec((blk, DH), lambda i: (i, 0)),
            pl.BlockSpec((blk, DH), lambda i: (i, 0)),
            pl.BlockSpec((8, DH), lambda i: (0, 0)),
            pl.BlockSpec((1, DH), lambda i: (0, 0)),
            pl.BlockSpec((1, DH), lambda i: (0, 0)),
        ],
        out_specs=pl.BlockSpec((blk, DH), lambda i: (i, 0)),
        out_shape=jax.ShapeDtypeStruct((NN, DH), jnp.float32),
    )(aggn, h, stats, g2, t2)


def _pool_body(h_ref, b_ref, w1_ref, b1_ref, w2_ref, b2_ref, o_ref, acc):
    i = pl.program_id(0)

    @pl.when(i == 0)
    def _():
        acc[...] = jnp.zeros_like(acc)

    bvec = b_ref[0, 0, :]
    onehot = (bvec[:, None] == lax.broadcasted_iota(jnp.int32, (bvec.shape[0], BB), 1)).astype(jnp.float32)
    acc[...] += lax.dot_general(onehot, h_ref[...], (((0,), (0,)), ((), ())),
                                preferred_element_type=jnp.float32)

    @pl.when(i == pl.num_programs(0) - 1)
    def _():
        hid = _softplus(
            jnp.dot(acc[...], w1_ref[...], preferred_element_type=jnp.float32)
            + b1_ref[...])
        o_ref[...] = jnp.dot(hid, w2_ref[...], preferred_element_type=jnp.float32) + b2_ref[...]


def _pool_mlp(h, batch3, W1, b1_2, W2, b2_2):
    blk = 2000
    grid = NN // blk
    return pl.pallas_call(
        _pool_body,
        grid=(grid,),
        in_specs=[
            pl.BlockSpec((blk, DH), lambda i: (i, 0)),
            pl.BlockSpec((1, 1, blk), lambda i: (i, 0, 0)),
            pl.BlockSpec((DH, HFEA), lambda i: (0, 0)),
            pl.BlockSpec((1, HFEA), lambda i: (0, 0)),
            pl.BlockSpec((HFEA, 1), lambda i: (0, 0)),
            pl.BlockSpec((1, 1), lambda i: (0, 0)),
        ],
        out_specs=pl.BlockSpec((BB, 1), lambda i: (0, 0)),
        out_shape=jax.ShapeDtypeStruct((BB, 1), jnp.float32),
        scratch_shapes=[pltpu.VMEM((BB, DH), jnp.float32)],
    )(h, batch3, W1, b1_2, W2, b2_2)


NCHW = EP // 16 // CH


def _sc_mesh():
    return plsc.VectorSubcoreMesh(core_axis_name="c", subcore_axis_name="s",
                                  num_cores=2, num_subcores=16)


_SC_PARAMS = pltpu.CompilerParams(use_tc_tiling_on_sc=False)


GB = 2
NBATCHG = NCHW // GB


def _gather_flow(tab_hbm, idx3_hbm, out_hbm, lane, idx_v, rows, gsem, wsem, nchw):
    nbatch = nchw // GB
    base = lane * (nchw * CH)
    pltpu.sync_copy(idx3_hbm.at[lane], idx_v)

    def g_issue(cc, slot, sem):
        pltpu.async_copy(tab_hbm.at[idx_v.at[cc]], rows.at[slot], sem)

    def g_wait(slot, sem):
        pltpu.make_async_copy(tab_hbm.at[idx_v.at[0]], rows.at[slot], sem).wait()

    def w_issue(cc, slot, sem):
        pltpu.async_copy(rows.at[slot], out_hbm.at[pl.ds(base + cc * CH, CH)], sem)

    def w_wait(slot, sem):
        pltpu.make_async_copy(rows.at[slot], out_hbm.at[pl.ds(base, CH)], sem).wait()

    for k in range(GB):
        g_issue(k, k, gsem.at[0])

    def body(b0, _):
        for hb in range(2):
            bb = 2 * b0 + hb
            nb = bb + 1

            @pl.when(bb >= 1)
            def _():
                for k in range(GB):
                    w_wait((1 - hb) * GB + k, wsem.at[1 - hb])

            @pl.when(nb < nbatch)
            def _():
                for k in range(GB):
                    g_issue(nb * GB + k, (1 - hb) * GB + k, gsem.at[1 - hb])

            for k in range(GB):
                g_wait(hb * GB + k, gsem.at[hb])
            for k in range(GB):
                w_issue(bb * GB + k, hb * GB + k, wsem.at[hb])
        return _

    lax.fori_loop(0, nbatch // 2, body, None)
    for k in range(GB):
        w_wait(GB + k, wsem.at[1])


def _sc_gather(pd, ps, dst3, src3):
    mesh = _sc_mesh()
    nchw = dst3.shape[1]
    eps = 16 * nchw * CH

    @functools.partial(
        pl.kernel,
        out_type=(jax.ShapeDtypeStruct((eps, 2 * DH), jnp.float32),
                  jax.ShapeDtypeStruct((eps, 2 * DH), jnp.float32)),
        mesh=mesh,
        scratch_types=[
            pltpu.VMEM((nchw, CH), jnp.int32),
            pltpu.VMEM((2 * GB, CH, 2 * DH), jnp.float32),
            pltpu.SemaphoreType.DMA((2,)),
            pltpu.SemaphoreType.DMA((2,)),
        ],
        compiler_params=_SC_PARAMS,
    )
    def k(pd_hbm, ps_hbm, dst3_hbm, src3_hbm, gd_hbm, gs_hbm, idx_v, rows, gsem, wsem):
        wid = lax.axis_index("s") * 2 + lax.axis_index("c")
        lane = wid % 16

        @pl.when(wid < 16)
        def _():
            _gather_flow(pd_hbm, dst3_hbm, gd_hbm, lane, idx_v, rows, gsem, wsem, nchw)

        @pl.when(wid >= 16)
        def _():
            _gather_flow(ps_hbm, src3_hbm, gs_hbm, lane, idx_v, rows, gsem, wsem, nchw)

    return k(pd, ps, dst3, src3)


NHALF = NPAD // 2
ROWS_PT = NHALF // 16
ZCH = 64
SENT = -1


def _scatter_flow(m_hbm, idx3_hbm, out_hbm, tile, idxr, rows, zbuf, acc,
                  lsem, asem, zsem, nchw):
    def zrow(r, _):
        for j in range(4):
            zbuf[r, pl.ds(16 * j, 16)] = jnp.zeros((16,), jnp.float32)
        return _
    lax.fori_loop(0, ZCH, zrow, None)
    for q in range(ROWS_PT // ZCH):
        pltpu.async_copy(zbuf, acc.at[pl.ds(tile * ROWS_PT + q * ZCH, ZCH)], zsem)
    for q in range(ROWS_PT // ZCH):
        pltpu.make_async_copy(zbuf, acc.at[pl.ds(0, ZCH)], zsem).wait()
    plsc.subcore_barrier()

    mbase = tile * (nchw * CH // 2)

    def l_issue(cc, slot, sem):
        r0 = mbase + cc * (CH // 2)
        pltpu.async_copy(m_hbm.at[pl.ds(r0, CH // 2), pl.ds(0, DH)],
                         rows.at[slot].at[pl.ds(0, CH // 2)], sem)
        pltpu.async_copy(m_hbm.at[pl.ds(r0, CH // 2), pl.ds(DH, DH)],
                         rows.at[slot].at[pl.ds(CH // 2, CH // 2)], sem)
        pltpu.async_copy(idx3_hbm.at[tile].at[cc], idxr.at[slot], sem)

    def l_wait(slot, sem):
        pltpu.make_async_copy(m_hbm.at[pl.ds(mbase, CH // 2), pl.ds(0, DH)],
                              rows.at[slot].at[pl.ds(0, CH // 2)], sem).wait()
        pltpu.make_async_copy(m_hbm.at[pl.ds(mbase, CH // 2), pl.ds(DH, DH)],
                              rows.at[slot].at[pl.ds(CH // 2, CH // 2)], sem).wait()
        pltpu.make_async_copy(idx3_hbm.at[tile].at[0], idxr.at[slot], sem).wait()

    def a_issue(slot, sem):
        pltpu.async_copy(rows.at[slot],
                         acc.at[plsc.Indices(idxr.at[slot], ignored_value=SENT)],
                         sem, add=True)

    def a_wait(slot, sem):
        pltpu.make_async_copy(rows.at[slot],
                              acc.at[plsc.Indices(idxr.at[slot], ignored_value=SENT)],
                              sem).wait()

    l_issue(0, 0, lsem.at[0])

    def body(b0, _):
        for hb in range(2):
            bb = 2 * b0 + hb
            nb = bb + 1

            @pl.when(bb >= 1)
            def _():
                a_wait(1 - hb, asem.at[1 - hb])

            @pl.when(nb < nchw)
            def _():
                l_issue(nb, 1 - hb, lsem.at[1 - hb])

            l_wait(hb, lsem.at[hb])
            a_issue(hb, asem.at[hb])
        return _

    lax.fori_loop(0, nchw // 2, body, None)
    a_wait(1, asem.at[1])
    plsc.subcore_barrier()
    pltpu.sync_copy(acc.at[pl.ds(tile * ROWS_PT, ROWS_PT)],
                    out_hbm.at[pl.ds(tile * ROWS_PT, ROWS_PT), pl.ds(0, DH)])


def _sc_scatter(m2, idxc0, idxc1):
    mesh = _sc_mesh()
    nchw = idxc0.shape[1]

    @functools.partial(
        pl.kernel,
        out_type=(jax.ShapeDtypeStruct((NHALF, 128), jnp.float32),
                  jax.ShapeDtypeStruct((NHALF, 128), jnp.float32)),
        mesh=mesh,
        scratch_types=[
            pltpu.VMEM((2, CH), jnp.int32),
            pltpu.VMEM((2, CH, DH), jnp.float32),
            pltpu.VMEM((ZCH, DH), jnp.float32),
            pltpu.VMEM_SHARED((NHALF, DH), jnp.float32),
            pltpu.SemaphoreType.DMA((2,)),
            pltpu.SemaphoreType.DMA((2,)),
            pltpu.SemaphoreType.DMA,
        ],
        compiler_params=_SC_PARAMS,
    )
    def k(m2_hbm, idxc0_hbm, idxc1_hbm, agg0_hbm, agg1_hbm,
          idxr, rows, zbuf, acc, lsem, asem, zsem):
        c = lax.axis_index("c")
        tile = lax.axis_index("s")

        @pl.when(c == 0)
        def _():
            _scatter_flow(m2_hbm, idxc0_hbm, agg0_hbm, tile, idxr, rows, zbuf,
                          acc, lsem, asem, zsem, nchw)

        @pl.when(c == 1)
        def _():
            _scatter_flow(m2_hbm, idxc1_hbm, agg1_hbm, tile, idxr, rows, zbuf,
                          acc, lsem, asem, zsem, nchw)

    return k(m2, idxc0, idxc1)


def _gather_edges(pd, ps, dst3, src3):
    return _sc_gather(pd, ps, dst3, src3)


def _scatter_edges(m2, idxc0, idxc1):
    return _sc_scatter(m2, idxc0, idxc1)


def kernel(x, edge_index, edge_attr, batch, We, be, Wf0, bf0, Ws0, bs0, g0, t0,
           Wf1, bf1, Ws1, bs1, g1, t1, Wf2, bf2, Ws2, bs2, g2, t2, W1, b1, W2, b2):
    src = edge_index[0]
    dst = edge_index[1]
    pad = EP - EE
    NSTRIP = 2
    ncs = NCHW // NSTRIP
    epss = EP // NSTRIP
    dst4 = jnp.pad(dst, (0, pad)).reshape(16, NSTRIP, ncs, CH)
    src4 = jnp.pad(src, (0, pad)).reshape(16, NSTRIP, ncs, CH)
    dd4 = jnp.pad(jnp.sum(edge_attr * edge_attr, axis=1), (0, pad)).reshape(16, NSTRIP, ncs * CH)
    dstp4 = jnp.pad(dst, (0, pad), constant_values=DUMMY).reshape(16, NSTRIP, ncs * CH)
    strips = []
    for s in range(NSTRIP):
        d3 = dst4[:, s]
        s3 = src4[:, s]
        dds = dd4[:, s].reshape(epss, 1)
        dsts = dstp4[:, s].reshape(epss)
        dperm = dsts.reshape(epss // 4096, 2, 32, 64).transpose(0, 2, 1, 3).reshape(epss)
        i0 = jnp.where(dperm < NHALF, dperm, SENT).astype(jnp.int32).reshape(16, ncs, CH)
        i1 = jnp.where(dperm >= NHALF, dperm - NHALF, SENT).astype(jnp.int32).reshape(16, ncs, CH)
        strips.append((d3, s3, dds, i0, i1))
    batch3 = batch.reshape(25, 1, 2000)

    h = _embed(x, We, be.reshape(1, DH))
    for (Wf, bf, Ws, bs, g, t) in ((Wf0, bf0, Ws0, bs0, g0, t0),
                                   (Wf1, bf1, Ws1, bs1, g1, t1),
                                   (Wf2, bf2, Ws2, bs2, g2, t2)):
        pd, ps = _project(h, Wf, Ws)
        aggs = []
        for (d3, s3, dds, i0, i1) in strips:
            gd, gs = _gather_edges(pd, ps, d3, s3)
            m2 = _edge_gate(gd, gs, dds, Wf, bf.reshape(1, DH), Ws, bs.reshape(1, DH))
            aggs.append(_scatter_edges(m2, i0, i1))
        h = _bn_residual(aggs, h, g.reshape(1, DH), t.reshape(1, DH))

    return _pool_mlp(h, batch3, W1, b1.reshape(1, HFEA), W2, b2.reshape(1, 1))
